# baseline (speedup 1.0000x reference)
import functools

import jax
import jax.numpy as jnp
from jax import lax
from jax.experimental import pallas as pl
from jax.experimental.pallas import tpu as pltpu
from jax.experimental.pallas import tpu_sc as plsc

K = 8192
D = 64
BETA = 0.25

BN = 256
KB = 2048
N_TOK = 16 * 576


def _dist_argmin_body(z_ref, zn_ref, cb_ref, cn_ref, kidx_ref, loss_ref):
    i = pl.program_id(0)
    z = z_ref[...]
    zn = zn_ref[...]
    best = None
    bestidx = None
    lossval = None
    iota = lax.broadcasted_iota(jnp.int32, (BN, KB), 1)
    for j in range(K // KB):
        cb_chunk = cb_ref[pl.ds(j * KB, KB), :]
        m = lax.dot_general(z, cb_chunk, (((1,), (1,)), ((), ())),
                            preferred_element_type=jnp.float32)
        cn_chunk = cn_ref[:, pl.ds(j * KB, KB)]
        d = (zn + m) + cn_chunk
        mind = jnp.min(d, axis=1, keepdims=True)
        midx = jnp.min(jnp.where(d == mind, iota, jnp.int32(2**30)),
                       axis=1, keepdims=True) + (j * KB)
        if best is None:
            bestidx, lossval = midx, mind
        else:
            take = mind < best
            bestidx = jnp.where(take, midx, bestidx)
            lossval = jnp.where(take, mind, lossval)
        nb = mind if best is None else jnp.where(take, mind, best)
        best = nb.astype(jnp.bfloat16).astype(jnp.float32)
    kidx_ref[...] = bestidx

    @pl.when(i == 0)
    def _init():
        loss_ref[...] = jnp.zeros((1, 1), jnp.float32)

    loss_ref[...] += jnp.sum(lossval, keepdims=True) * (BETA / (N_TOK * D))


@functools.partial(jax.jit, static_argnums=())
def _dist_argmin(z, zn, codebook, cn):
    n = z.shape[0]
    grid = (n // BN,)
    return pl.pallas_call(
        _dist_argmin_body,
        grid=grid,
        in_specs=[
            pl.BlockSpec((BN, D), lambda i: (i, 0)),
            pl.BlockSpec((BN, 1), lambda i: (i, 0)),
            pl.BlockSpec((K, D), lambda i: (0, 0)),
            pl.BlockSpec((1, K), lambda i: (0, 0)),
        ],
        out_specs=[
            pl.BlockSpec((BN, 1), lambda i: (i, 0)),
            pl.BlockSpec((1, 1), lambda i: (0, 0)),
        ],
        out_shape=[
            jax.ShapeDtypeStruct((n, 1), jnp.int32),
            jax.ShapeDtypeStruct((1, 1), jnp.float32),
        ],
    )(z, zn, codebook, cn)


NW = 32
TPW = (16 * 576) // NW
CPW = K // NW


def _sc_body(cb_hbm, k_hbm, z_hbm, zeros_hbm, ones_hbm, out_hbm, util_hbm,
             idx_v, rows_v, zrows_v, idx576_v, ones576_v, cntbuf_v, cnt_v,
             shared_v, sem):
    c = lax.axis_index("c")
    s = lax.axis_index("s")
    wid = s * 2 + c
    base = wid * TPW
    pltpu.sync_copy(k_hbm.at[pl.ds(base, TPW)], idx_v)
    pltpu.async_copy(cb_hbm.at[idx_v], rows_v, sem).wait()
    pltpu.sync_copy(z_hbm.at[pl.ds(base, TPW)], zrows_v)

    def _st_row(r, carry):
        for cc in range(D // 16):
            sl = pl.ds(cc * 16, 16)
            zq = rows_v[r, sl]
            zz = zrows_v[r, sl]
            rows_v[r, sl] = zz + (zq - zz)
        return carry

    lax.fori_loop(0, TPW, _st_row, 0, unroll=4)
    pltpu.sync_copy(rows_v, out_hbm.at[pl.ds(base, TPW)])

    pltpu.sync_copy(zeros_hbm, shared_v.at[pl.ds(s * (K // 16), K // 16)])
    pltpu.sync_copy(ones_hbm, ones576_v)
    pltpu.sync_copy(k_hbm.at[pl.ds(s * (N_TOK // 16), N_TOK // 16)], idx576_v)
    plsc.subcore_barrier()
    pltpu.sync_copy(ones576_v, shared_v.at[idx576_v], add=True)
    plsc.subcore_barrier()
    r0 = c * (K // 2) + s * CPW
    pltpu.sync_copy(shared_v.at[pl.ds(r0, CPW)], cntbuf_v)

    def _count(i, acc):
        f = cntbuf_v[i, :]
        return acc + jnp.where(f > 0.0, 1.0, 0.0)

    acc = lax.fori_loop(0, CPW, _count, jnp.zeros((16,), jnp.float32),
                        unroll=4)
    cnt_v[...] = acc
    pltpu.sync_copy(cnt_v, util_hbm.at[wid])


@jax.jit
def _sc_gather_util(codebook, k, z, zeros512, ones576):
    n = k.shape[0]
    mesh = plsc.VectorSubcoreMesh(core_axis_name="c", subcore_axis_name="s")
    f = pl.kernel(
        _sc_body,
        out_type=[
            jax.ShapeDtypeStruct((n, D), jnp.float32),
            jax.ShapeDtypeStruct((NW, 16), jnp.float32),
        ],
        mesh=mesh,
        compiler_params=pltpu.CompilerParams(use_tc_tiling_on_sc=False),
        scratch_types=[
            pltpu.VMEM((TPW,), jnp.int32),
            pltpu.VMEM((TPW, D), jnp.float32),
            pltpu.VMEM((TPW, D), jnp.float32),
            pltpu.VMEM((N_TOK // 16,), jnp.int32),
            pltpu.VMEM((N_TOK // 16, 16), jnp.float32),
            pltpu.VMEM((CPW, 16), jnp.float32),
            pltpu.VMEM((16,), jnp.float32),
            pltpu.VMEM_SHARED((K, 16), jnp.float32),
            pltpu.SemaphoreType.DMA,
        ],
    )
    return f(codebook, k, z, zeros512, ones576)


def kernel(z_e, codebook):
    B, T, Dd = z_e.shape
    z = z_e.reshape(B * T, Dd)
    zn = jnp.sum(z ** 2, axis=1, keepdims=True)
    cn = jnp.sum(codebook ** 2, axis=1).reshape(1, K)
    cm2 = -2.0 * codebook
    kidx2, loss = _dist_argmin(z, zn, cm2, cn)
    k = kidx2[:, 0]
    zeros512 = jnp.zeros((K // 16, 16), jnp.float32)
    ones576 = jnp.ones((N_TOK // 16, 16), jnp.float32)
    z_q_st, util_parts = _sc_gather_util(codebook, k, z, zeros512, ones576)
    utilization = jnp.sum(util_parts) * (1.0 / (16 * K))
    return (z_q_st.reshape(B, T, Dd), k.reshape(B, T), loss[0, 0], utilization)


# speedup vs baseline: 1.0545x; 1.0545x over previous
import functools

import jax
import jax.numpy as jnp
from jax import lax
from jax.experimental import pallas as pl
from jax.experimental.pallas import tpu as pltpu
from jax.experimental.pallas import tpu_sc as plsc

K = 8192
D = 64
BETA = 0.25

BN = 256
KB = 2048
N_TOK = 16 * 576


def _dist_argmin_body(z_ref, zn_ref, cb_ref, cn_ref, kidx_ref, loss_ref):
    i = pl.program_id(0)
    z = z_ref[...]
    zn = zn_ref[...]
    best = None
    bestidx = None
    lossval = None
    iota = lax.broadcasted_iota(jnp.int32, (BN, KB), 1)
    for j in range(K // KB):
        cb_chunk = cb_ref[pl.ds(j * KB, KB), :]
        m = lax.dot_general(z, cb_chunk, (((1,), (1,)), ((), ())),
                            preferred_element_type=jnp.float32)
        cn_chunk = cn_ref[:, pl.ds(j * KB, KB)]
        d = (zn + m) + cn_chunk
        mind = jnp.min(d, axis=1, keepdims=True)
        midx = jnp.min(jnp.where(d == mind, iota, jnp.int32(2**30)),
                       axis=1, keepdims=True) + (j * KB)
        if best is None:
            bestidx, lossval = midx, mind
        else:
            take = mind < best
            bestidx = jnp.where(take, midx, bestidx)
            lossval = jnp.where(take, mind, lossval)
        nb = mind if best is None else jnp.where(take, mind, best)
        best = nb.astype(jnp.bfloat16).astype(jnp.float32)
    kidx_ref[...] = bestidx

    @pl.when(i == 0)
    def _init():
        loss_ref[...] = jnp.zeros((1, 1), jnp.float32)

    loss_ref[...] += jnp.sum(lossval, keepdims=True) * (BETA / (N_TOK * D))


@functools.partial(jax.jit, static_argnums=())
def _dist_argmin(z, zn, codebook, cn):
    n = z.shape[0]
    grid = (n // BN,)
    return pl.pallas_call(
        _dist_argmin_body,
        grid=grid,
        in_specs=[
            pl.BlockSpec((BN, D), lambda i: (i, 0)),
            pl.BlockSpec((BN, 1), lambda i: (i, 0)),
            pl.BlockSpec((K, D), lambda i: (0, 0)),
            pl.BlockSpec((1, K), lambda i: (0, 0)),
        ],
        out_specs=[
            pl.BlockSpec((BN, 1), lambda i: (i, 0)),
            pl.BlockSpec((1, 1), lambda i: (0, 0)),
        ],
        out_shape=[
            jax.ShapeDtypeStruct((n, 1), jnp.int32),
            jax.ShapeDtypeStruct((1, 1), jnp.float32),
        ],
    )(z, zn, codebook, cn)


NW = 32
TPW = (16 * 576) // NW
CPW = K // NW


def _sc_body(cb_hbm, k_hbm, zeros_hbm, ones_hbm, out_hbm, util_hbm,
             idx_v, rows_v, idx576_v, ones576_v, cntbuf_v, cnt_v,
             shared_v, sem):
    c = lax.axis_index("c")
    s = lax.axis_index("s")
    wid = s * 2 + c
    base = wid * TPW
    pltpu.sync_copy(k_hbm.at[pl.ds(base, TPW)], idx_v)
    pltpu.async_copy(cb_hbm.at[idx_v], rows_v, sem).wait()
    pltpu.sync_copy(rows_v, out_hbm.at[pl.ds(base, TPW)])

    pltpu.sync_copy(zeros_hbm, shared_v.at[pl.ds(s * (K // 16), K // 16)])
    pltpu.sync_copy(ones_hbm, ones576_v)
    pltpu.sync_copy(k_hbm.at[pl.ds(s * (N_TOK // 16), N_TOK // 16)], idx576_v)
    plsc.subcore_barrier()
    pltpu.sync_copy(ones576_v, shared_v.at[idx576_v], add=True)
    plsc.subcore_barrier()
    r0 = c * (K // 2) + s * CPW
    pltpu.sync_copy(shared_v.at[pl.ds(r0, CPW)], cntbuf_v)

    def _count(i, acc):
        f = cntbuf_v[i, :]
        return acc + jnp.where(f > 0.0, 1.0, 0.0)

    acc = lax.fori_loop(0, CPW, _count, jnp.zeros((16,), jnp.float32),
                        unroll=4)
    cnt_v[...] = acc
    pltpu.sync_copy(cnt_v, util_hbm.at[wid])


@jax.jit
def _sc_gather_util(codebook, k, zeros512, ones576):
    n = k.shape[0]
    mesh = plsc.VectorSubcoreMesh(core_axis_name="c", subcore_axis_name="s")
    f = pl.kernel(
        _sc_body,
        out_type=[
            jax.ShapeDtypeStruct((n, D), jnp.float32),
            jax.ShapeDtypeStruct((NW, 16), jnp.float32),
        ],
        mesh=mesh,
        compiler_params=pltpu.CompilerParams(use_tc_tiling_on_sc=False),
        scratch_types=[
            pltpu.VMEM((TPW,), jnp.int32),
            pltpu.VMEM((TPW, D), jnp.float32),
            pltpu.VMEM((N_TOK // 16,), jnp.int32),
            pltpu.VMEM((N_TOK // 16, 16), jnp.float32),
            pltpu.VMEM((CPW, 16), jnp.float32),
            pltpu.VMEM((16,), jnp.float32),
            pltpu.VMEM_SHARED((K, 16), jnp.float32),
            pltpu.SemaphoreType.DMA,
        ],
    )
    return f(codebook, k, zeros512, ones576)


def kernel(z_e, codebook):
    B, T, Dd = z_e.shape
    z = z_e.reshape(B * T, Dd)
    zn = jnp.sum(z ** 2, axis=1, keepdims=True)
    cn = jnp.sum(codebook ** 2, axis=1).reshape(1, K)
    cm2 = -2.0 * codebook
    kidx2, loss = _dist_argmin(z, zn, cm2, cn)
    k = kidx2[:, 0]
    zeros512 = jnp.zeros((K // 16, 16), jnp.float32)
    ones576 = jnp.ones((N_TOK // 16, 16), jnp.float32)
    z_q, util_parts = _sc_gather_util(codebook, k, zeros512, ones576)
    z_q_st = z + lax.stop_gradient(z_q - z)
    utilization = jnp.sum(util_parts) * (1.0 / (16 * K))
    return (z_q_st.reshape(B, T, Dd), k.reshape(B, T), loss[0, 0], utilization)


# speedup vs baseline: 1.1036x; 1.0466x over previous
import functools

import jax
import jax.numpy as jnp
from jax import lax
from jax.experimental import pallas as pl
from jax.experimental.pallas import tpu as pltpu
from jax.experimental.pallas import tpu_sc as plsc

K = 8192
D = 64
BETA = 0.25

BN = 512
KB = 2048
N_TOK = 16 * 576


def _dist_argmin_body(z_ref, zn_ref, cb_ref, cn_ref, kidx_ref, loss_ref):
    i = pl.program_id(0)
    z = z_ref[...]
    zn = zn_ref[...]
    best = None
    bestidx = None
    lossval = None
    iota = lax.broadcasted_iota(jnp.int32, (BN, KB), 1)
    for j in range(K // KB):
        cb_chunk = cb_ref[pl.ds(j * KB, KB), :]
        m = lax.dot_general(z, cb_chunk, (((1,), (1,)), ((), ())),
                            preferred_element_type=jnp.float32)
        cn_chunk = cn_ref[:, pl.ds(j * KB, KB)]
        d = (zn + m) + cn_chunk
        mind = jnp.min(d, axis=1, keepdims=True)
        midx = jnp.min(jnp.where(d == mind, iota, jnp.int32(2**30)),
                       axis=1, keepdims=True) + (j * KB)
        if best is None:
            bestidx, lossval = midx, mind
        else:
            take = mind < best
            bestidx = jnp.where(take, midx, bestidx)
            lossval = jnp.where(take, mind, lossval)
        nb = mind if best is None else jnp.where(take, mind, best)
        best = nb.astype(jnp.bfloat16).astype(jnp.float32)
    kidx_ref[...] = bestidx

    @pl.when(i == 0)
    def _init():
        loss_ref[...] = jnp.zeros((1, 1), jnp.float32)

    loss_ref[...] += jnp.sum(lossval, keepdims=True) * (BETA / (N_TOK * D))


@functools.partial(jax.jit, static_argnums=())
def _dist_argmin(z, zn, codebook, cn):
    n = z.shape[0]
    grid = (n // BN,)
    return pl.pallas_call(
        _dist_argmin_body,
        grid=grid,
        in_specs=[
            pl.BlockSpec((BN, D), lambda i: (i, 0)),
            pl.BlockSpec((BN, 1), lambda i: (i, 0)),
            pl.BlockSpec((K, D), lambda i: (0, 0)),
            pl.BlockSpec((1, K), lambda i: (0, 0)),
        ],
        out_specs=[
            pl.BlockSpec((BN, 1), lambda i: (i, 0)),
            pl.BlockSpec((1, 1), lambda i: (0, 0)),
        ],
        out_shape=[
            jax.ShapeDtypeStruct((n, 1), jnp.int32),
            jax.ShapeDtypeStruct((1, 1), jnp.float32),
        ],
    )(z, zn, codebook, cn)


NW = 32
TPW = (16 * 576) // NW
CPW = K // NW


def _sc_body(cb_hbm, k_hbm, zeros_hbm, ones_hbm, out_hbm, util_hbm,
             idx_v, rows_v, idx576_v, ones576_v, cntbuf_v, cnt_v,
             shared_v, sem):
    c = lax.axis_index("c")
    s = lax.axis_index("s")
    wid = s * 2 + c
    base = wid * TPW
    pltpu.sync_copy(k_hbm.at[pl.ds(base, TPW)], idx_v)
    pltpu.async_copy(cb_hbm.at[idx_v], rows_v, sem).wait()
    pltpu.sync_copy(rows_v, out_hbm.at[pl.ds(base, TPW)])

    pltpu.sync_copy(zeros_hbm, shared_v.at[pl.ds(s * (K // 16), K // 16)])
    pltpu.sync_copy(ones_hbm, ones576_v)
    pltpu.sync_copy(k_hbm.at[pl.ds(s * (N_TOK // 16), N_TOK // 16)], idx576_v)
    plsc.subcore_barrier()
    pltpu.sync_copy(ones576_v, shared_v.at[idx576_v], add=True)
    plsc.subcore_barrier()
    r0 = c * (K // 2) + s * CPW
    pltpu.sync_copy(shared_v.at[pl.ds(r0, CPW)], cntbuf_v)

    def _count(i, acc):
        f = cntbuf_v[i, :]
        return acc + jnp.where(f > 0.0, 1.0, 0.0)

    acc = lax.fori_loop(0, CPW, _count, jnp.zeros((16,), jnp.float32),
                        unroll=4)
    cnt_v[...] = acc
    pltpu.sync_copy(cnt_v, util_hbm.at[wid])


@jax.jit
def _sc_gather_util(codebook, k, zeros512, ones576):
    n = k.shape[0]
    mesh = plsc.VectorSubcoreMesh(core_axis_name="c", subcore_axis_name="s")
    f = pl.kernel(
        _sc_body,
        out_type=[
            jax.ShapeDtypeStruct((n, D), jnp.float32),
            jax.ShapeDtypeStruct((NW, 16), jnp.float32),
        ],
        mesh=mesh,
        compiler_params=pltpu.CompilerParams(use_tc_tiling_on_sc=False),
        scratch_types=[
            pltpu.VMEM((TPW,), jnp.int32),
            pltpu.VMEM((TPW, D), jnp.float32),
            pltpu.VMEM((N_TOK // 16,), jnp.int32),
            pltpu.VMEM((N_TOK // 16, 16), jnp.float32),
            pltpu.VMEM((CPW, 16), jnp.float32),
            pltpu.VMEM((16,), jnp.float32),
            pltpu.VMEM_SHARED((K, 16), jnp.float32),
            pltpu.SemaphoreType.DMA,
        ],
    )
    return f(codebook, k, zeros512, ones576)


def kernel(z_e, codebook):
    B, T, Dd = z_e.shape
    z = z_e.reshape(B * T, Dd)
    zn = jnp.sum(z ** 2, axis=1, keepdims=True)
    cn = jnp.sum(codebook ** 2, axis=1).reshape(1, K)
    cm2 = -2.0 * codebook
    kidx2, loss = _dist_argmin(z, zn, cm2, cn)
    k = kidx2[:, 0]
    zeros512 = jnp.zeros((K // 16, 16), jnp.float32)
    ones576 = jnp.ones((N_TOK // 16, 16), jnp.float32)
    z_q, util_parts = _sc_gather_util(codebook, k, zeros512, ones576)
    z_q_st = z + lax.stop_gradient(z_q - z)
    utilization = jnp.sum(util_parts) * (1.0 / (16 * K))
    return (z_q_st.reshape(B, T, Dd), k.reshape(B, T), loss[0, 0], utilization)


# speedup vs baseline: 1.1241x; 1.0185x over previous
import functools

import jax
import jax.numpy as jnp
from jax import lax
from jax.experimental import pallas as pl
from jax.experimental.pallas import tpu as pltpu
from jax.experimental.pallas import tpu_sc as plsc

K = 8192
D = 64
BETA = 0.25

BN = 1024
KB = 2048
N_TOK = 16 * 576


def _dist_argmin_body(z_ref, zn_ref, cb_ref, cn_ref, kidx_ref, loss_ref):
    i = pl.program_id(0)
    z = z_ref[...]
    zn = zn_ref[...]
    best = None
    bestidx = None
    lossval = None
    iota = lax.broadcasted_iota(jnp.int32, (BN, KB), 1)
    for j in range(K // KB):
        cb_chunk = cb_ref[pl.ds(j * KB, KB), :]
        m = lax.dot_general(z, cb_chunk, (((1,), (1,)), ((), ())),
                            preferred_element_type=jnp.float32)
        cn_chunk = cn_ref[:, pl.ds(j * KB, KB)]
        d = (zn + m) + cn_chunk
        mind = jnp.min(d, axis=1, keepdims=True)
        midx = jnp.min(jnp.where(d == mind, iota, jnp.int32(2**30)),
                       axis=1, keepdims=True) + (j * KB)
        if best is None:
            bestidx, lossval = midx, mind
        else:
            take = mind < best
            bestidx = jnp.where(take, midx, bestidx)
            lossval = jnp.where(take, mind, lossval)
        nb = mind if best is None else jnp.where(take, mind, best)
        best = nb.astype(jnp.bfloat16).astype(jnp.float32)
    kidx_ref[...] = bestidx

    @pl.when(i == 0)
    def _init():
        loss_ref[...] = jnp.zeros((1, 1), jnp.float32)

    loss_ref[...] += jnp.sum(lossval, keepdims=True) * (BETA / (N_TOK * D))


@functools.partial(jax.jit, static_argnums=())
def _dist_argmin(z, zn, codebook, cn):
    n = z.shape[0]
    grid = (n // BN,)
    return pl.pallas_call(
        _dist_argmin_body,
        grid=grid,
        in_specs=[
            pl.BlockSpec((BN, D), lambda i: (i, 0)),
            pl.BlockSpec((BN, 1), lambda i: (i, 0)),
            pl.BlockSpec((K, D), lambda i: (0, 0)),
            pl.BlockSpec((1, K), lambda i: (0, 0)),
        ],
        out_specs=[
            pl.BlockSpec((BN, 1), lambda i: (i, 0)),
            pl.BlockSpec((1, 1), lambda i: (0, 0)),
        ],
        out_shape=[
            jax.ShapeDtypeStruct((n, 1), jnp.int32),
            jax.ShapeDtypeStruct((1, 1), jnp.float32),
        ],
    )(z, zn, codebook, cn)


NW = 32
TPW = (16 * 576) // NW
CPW = K // NW


def _sc_body(cb_hbm, k_hbm, zeros_hbm, ones_hbm, out_hbm, util_hbm,
             idx_v, rows_v, idx576_v, ones576_v, cntbuf_v, cnt_v,
             shared_v, sem):
    c = lax.axis_index("c")
    s = lax.axis_index("s")
    wid = s * 2 + c
    base = wid * TPW
    pltpu.sync_copy(k_hbm.at[pl.ds(base, TPW)], idx_v)
    pltpu.async_copy(cb_hbm.at[idx_v], rows_v, sem).wait()
    pltpu.sync_copy(rows_v, out_hbm.at[pl.ds(base, TPW)])

    pltpu.sync_copy(zeros_hbm, shared_v.at[pl.ds(s * (K // 16), K // 16)])
    pltpu.sync_copy(ones_hbm, ones576_v)
    pltpu.sync_copy(k_hbm.at[pl.ds(s * (N_TOK // 16), N_TOK // 16)], idx576_v)
    plsc.subcore_barrier()
    pltpu.sync_copy(ones576_v, shared_v.at[idx576_v], add=True)
    plsc.subcore_barrier()
    r0 = c * (K // 2) + s * CPW
    pltpu.sync_copy(shared_v.at[pl.ds(r0, CPW)], cntbuf_v)

    def _count(i, acc):
        f = cntbuf_v[i, :]
        return acc + jnp.where(f > 0.0, 1.0, 0.0)

    acc = lax.fori_loop(0, CPW, _count, jnp.zeros((16,), jnp.float32),
                        unroll=4)
    cnt_v[...] = acc
    pltpu.sync_copy(cnt_v, util_hbm.at[wid])


@jax.jit
def _sc_gather_util(codebook, k, zeros512, ones576):
    n = k.shape[0]
    mesh = plsc.VectorSubcoreMesh(core_axis_name="c", subcore_axis_name="s")
    f = pl.kernel(
        _sc_body,
        out_type=[
            jax.ShapeDtypeStruct((n, D), jnp.float32),
            jax.ShapeDtypeStruct((NW, 16), jnp.float32),
        ],
        mesh=mesh,
        compiler_params=pltpu.CompilerParams(use_tc_tiling_on_sc=False),
        scratch_types=[
            pltpu.VMEM((TPW,), jnp.int32),
            pltpu.VMEM((TPW, D), jnp.float32),
            pltpu.VMEM((N_TOK // 16,), jnp.int32),
            pltpu.VMEM((N_TOK // 16, 16), jnp.float32),
            pltpu.VMEM((CPW, 16), jnp.float32),
            pltpu.VMEM((16,), jnp.float32),
            pltpu.VMEM_SHARED((K, 16), jnp.float32),
            pltpu.SemaphoreType.DMA,
        ],
    )
    return f(codebook, k, zeros512, ones576)


def kernel(z_e, codebook):
    B, T, Dd = z_e.shape
    z = z_e.reshape(B * T, Dd)
    zn = jnp.sum(z ** 2, axis=1, keepdims=True)
    cn = jnp.sum(codebook ** 2, axis=1).reshape(1, K)
    cm2 = -2.0 * codebook
    kidx2, loss = _dist_argmin(z, zn, cm2, cn)
    k = kidx2[:, 0]
    zeros512 = jnp.zeros((K // 16, 16), jnp.float32)
    ones576 = jnp.ones((N_TOK // 16, 16), jnp.float32)
    z_q, util_parts = _sc_gather_util(codebook, k, zeros512, ones576)
    z_q_st = z + lax.stop_gradient(z_q - z)
    utilization = jnp.sum(util_parts) * (1.0 / (16 * K))
    return (z_q_st.reshape(B, T, Dd), k.reshape(B, T), loss[0, 0], utilization)
